# baseline (speedup 1.0000x reference)
import jax
import jax.numpy as jnp
from jax.experimental import pallas as pl
from jax.experimental.pallas import tpu as pltpu

EPS_ = 1e-5
ROWS, COLS = 65536, 512
BLK = 1024
NB = ROWS // BLK


def _bn_kernel(x_ref, m_ref, g_ref, b_ref, o_ref,
               acc_s, acc_q, acc_c, coef_c, coef_b):
    p = pl.program_id(0)
    i = pl.program_id(1)

    @pl.when((p == 0) & (i == 0))
    def _init():
        acc_s[...] = jnp.zeros_like(acc_s)
        acc_q[...] = jnp.zeros_like(acc_q)
        acc_c[...] = jnp.zeros_like(acc_c)

    @pl.when(p == 0)
    def _accumulate():
        x = x_ref[...]
        m = m_ref[...]
        bm = jnp.broadcast_to(m, x.shape)
        xm = x * bm
        acc_s[...] += jnp.sum(xm, axis=0, keepdims=True)
        acc_q[...] += jnp.sum(xm * x, axis=0, keepdims=True)
        acc_c[...] += jnp.sum(bm, axis=0, keepdims=True)

    @pl.when((p == 1) & (i == 0))
    def _finalize():
        cnt = acc_c[...]
        mean = acc_s[...] / cnt
        var = acc_q[...] / cnt - mean * mean
        a = jax.lax.rsqrt(var + EPS_) * g_ref[...]
        coef_c[...] = a - 1.0
        coef_b[...] = b_ref[...] - mean * a

    @pl.when(p == 1)
    def _apply():
        x = x_ref[...]
        m = m_ref[...]
        t = x * coef_c[...] + coef_b[...]
        o_ref[...] = x + t * m


def kernel(x, mask, gamma, beta):
    m = mask.astype(jnp.float32).reshape(ROWS, 1)
    g = gamma.reshape(1, COLS)
    b = beta.reshape(1, COLS)
    out = pl.pallas_call(
        _bn_kernel,
        grid=(2, NB),
        in_specs=[
            pl.BlockSpec((BLK, COLS), lambda p, i: (i, 0)),
            pl.BlockSpec((BLK, 1), lambda p, i: (i, 0)),
            pl.BlockSpec((1, COLS), lambda p, i: (0, 0)),
            pl.BlockSpec((1, COLS), lambda p, i: (0, 0)),
        ],
        out_specs=pl.BlockSpec((BLK, COLS), lambda p, i: (p * i, 0)),
        out_shape=jax.ShapeDtypeStruct((ROWS, COLS), x.dtype),
        scratch_shapes=[
            pltpu.VMEM((1, COLS), jnp.float32),
            pltpu.VMEM((1, COLS), jnp.float32),
            pltpu.VMEM((1, COLS), jnp.float32),
            pltpu.VMEM((1, COLS), jnp.float32),
            pltpu.VMEM((1, COLS), jnp.float32),
        ],
    )(x, m, g, b)
    return out


# speedup vs baseline: 1.0014x; 1.0014x over previous
import jax
import jax.numpy as jnp
from jax.experimental import pallas as pl
from jax.experimental.pallas import tpu as pltpu

EPS_ = 1e-5
ROWS, COLS = 65536, 512
BLK = 1024
NB = ROWS // BLK


def _stats_kernel(x_ref, m_ref, s_ref, acc_s, acc_q, acc_c):
    i = pl.program_id(0)

    @pl.when(i == 0)
    def _init():
        acc_s[...] = jnp.zeros_like(acc_s)
        acc_q[...] = jnp.zeros_like(acc_q)
        acc_c[...] = jnp.zeros_like(acc_c)

    x = x_ref[...]
    m = m_ref[...]
    bm = jnp.broadcast_to(m, x.shape)
    xm = x * bm
    acc_s[...] += jnp.sum(xm, axis=0, keepdims=True)
    acc_q[...] += jnp.sum(xm * x, axis=0, keepdims=True)
    acc_c[...] += jnp.sum(m, axis=0, keepdims=True)

    @pl.when(i == NB - 1)
    def _emit():
        s_ref[0:1, :] = acc_s[...]
        s_ref[1:2, :] = acc_q[...]
        s_ref[2:3, :] = jnp.broadcast_to(acc_c[...], (1, COLS))


def _apply_kernel(x_ref, m_ref, s_ref, g_ref, b_ref, o_ref, coef_c, coef_b):
    i = pl.program_id(0)

    @pl.when(i == 0)
    def _finalize():
        cnt = s_ref[2:3, :]
        mean = s_ref[0:1, :] / cnt
        var = s_ref[1:2, :] / cnt - mean * mean
        a = jax.lax.rsqrt(var + EPS_) * g_ref[...]
        coef_c[...] = a - 1.0
        coef_b[...] = b_ref[...] - mean * a

    x = x_ref[...]
    m = m_ref[...]
    t = x * coef_c[...] + coef_b[...]
    o_ref[...] = x + t * m


def kernel(x, mask, gamma, beta):
    m = mask.astype(jnp.float32).reshape(ROWS, 1)
    g = gamma.reshape(1, COLS)
    b = beta.reshape(1, COLS)
    stats = pl.pallas_call(
        _stats_kernel,
        grid=(NB,),
        in_specs=[
            pl.BlockSpec((BLK, COLS), lambda i: (i, 0)),
            pl.BlockSpec((BLK, 1), lambda i: (i, 0)),
        ],
        out_specs=pl.BlockSpec((3, COLS), lambda i: (0, 0)),
        out_shape=jax.ShapeDtypeStruct((3, COLS), jnp.float32),
        scratch_shapes=[
            pltpu.VMEM((1, COLS), jnp.float32),
            pltpu.VMEM((1, COLS), jnp.float32),
            pltpu.VMEM((1, 1), jnp.float32),
        ],
    )(x, m)
    out = pl.pallas_call(
        _apply_kernel,
        grid=(NB,),
        in_specs=[
            pl.BlockSpec((BLK, COLS), lambda i: (i, 0)),
            pl.BlockSpec((BLK, 1), lambda i: (i, 0)),
            pl.BlockSpec((3, COLS), lambda i: (0, 0)),
            pl.BlockSpec((1, COLS), lambda i: (0, 0)),
            pl.BlockSpec((1, COLS), lambda i: (0, 0)),
        ],
        out_specs=pl.BlockSpec((BLK, COLS), lambda i: (i, 0)),
        out_shape=jax.ShapeDtypeStruct((ROWS, COLS), x.dtype),
        scratch_shapes=[
            pltpu.VMEM((1, COLS), jnp.float32),
            pltpu.VMEM((1, COLS), jnp.float32),
        ],
    )(x, m, stats, g, b)
    return out


# speedup vs baseline: 1.2074x; 1.2057x over previous
import jax
import jax.numpy as jnp
from jax.experimental import pallas as pl
from jax.experimental.pallas import tpu as pltpu

EPS_ = 1e-5
ROWS, COLS = 65536, 512
CH = 512
NCH = ROWS // CH
DEPTH = 8
NOUTER = NCH // DEPTH


def _bn_kernel(x_hbm, m_hbm, g_hbm, b_hbm, o_hbm,
               xbuf, mbuf, obuf, gloc, bloc,
               acc_s, acc_q, acc_c, coef_c, coef_b,
               sem_rx, sem_rm, sem_w, sem_misc):

    def read_x(j, s):
        return pltpu.make_async_copy(
            x_hbm.at[pl.ds(j * CH, CH), :], xbuf.at[s], sem_rx.at[s])

    def read_m(j, s):
        return pltpu.make_async_copy(
            m_hbm.at[pl.ds(j * CH, CH), :], mbuf.at[s], sem_rm.at[s])

    def write_o(j, s):
        return pltpu.make_async_copy(
            obuf.at[s], o_hbm.at[pl.ds(j * CH, CH), :], sem_w.at[s])

    cg = pltpu.make_async_copy(g_hbm, gloc, sem_misc.at[0])
    cb = pltpu.make_async_copy(b_hbm, bloc, sem_misc.at[1])
    cg.start()
    cb.start()

    acc_s[...] = jnp.zeros_like(acc_s)
    acc_q[...] = jnp.zeros_like(acc_q)
    acc_c[...] = jnp.zeros_like(acc_c)

    for s in range(DEPTH):
        read_x(s, s).start()
        read_m(s, s).start()

    def p0_body(j2, carry):
        for s in range(DEPTH):
            j = j2 * DEPTH + s
            read_x(j, s).wait()
            read_m(j, s).wait()
            x = xbuf[s]
            m = mbuf[s]
            xm = x * m
            acc_s[...] += jnp.sum(xm, axis=0, keepdims=True)
            acc_q[...] += jnp.sum(xm * xm, axis=0, keepdims=True)
            acc_c[...] += jnp.sum(m, axis=0, keepdims=True)

            @pl.when(j + DEPTH < NCH)
            def _():
                read_x(j + DEPTH, s).start()
                read_m(j + DEPTH, s).start()
        return carry

    jax.lax.fori_loop(0, NOUTER, p0_body, 0)

    cg.wait()
    cb.wait()
    cnt = jnp.broadcast_to(acc_c[...], (1, COLS))
    mean = acc_s[...] / cnt
    var = acc_q[...] / cnt - mean * mean
    a = jax.lax.rsqrt(var + EPS_) * gloc[...]
    coef_c[...] = a - 1.0
    coef_b[...] = bloc[...] - mean * a

    for s in range(DEPTH):
        read_x(s, s).start()
        read_m(s, s).start()

    def p1_body(j2, carry):
        for s in range(DEPTH):
            j = j2 * DEPTH + s
            read_x(j, s).wait()
            read_m(j, s).wait()

            @pl.when(j2 > 0)
            def _():
                write_o(j - DEPTH, s).wait()

            x = xbuf[s]
            m = mbuf[s]
            t = x * coef_c[...] + coef_b[...]
            obuf[s] = x + t * m
            write_o(j, s).start()

            @pl.when(j + DEPTH < NCH)
            def _():
                read_x(j + DEPTH, s).start()
                read_m(j + DEPTH, s).start()
        return carry

    jax.lax.fori_loop(0, NOUTER, p1_body, 0)

    for s in range(DEPTH):
        write_o(NCH - DEPTH + s, s).wait()


def kernel(x, mask, gamma, beta):
    m = mask.astype(jnp.float32).reshape(ROWS, 1)
    g = gamma.reshape(1, COLS)
    b = beta.reshape(1, COLS)
    out = pl.pallas_call(
        _bn_kernel,
        in_specs=[
            pl.BlockSpec(memory_space=pl.ANY),
            pl.BlockSpec(memory_space=pl.ANY),
            pl.BlockSpec(memory_space=pl.ANY),
            pl.BlockSpec(memory_space=pl.ANY),
        ],
        out_specs=pl.BlockSpec(memory_space=pl.ANY),
        out_shape=jax.ShapeDtypeStruct((ROWS, COLS), x.dtype),
        scratch_shapes=[
            pltpu.VMEM((DEPTH, CH, COLS), jnp.float32),
            pltpu.VMEM((DEPTH, CH, 1), jnp.float32),
            pltpu.VMEM((DEPTH, CH, COLS), jnp.float32),
            pltpu.VMEM((1, COLS), jnp.float32),
            pltpu.VMEM((1, COLS), jnp.float32),
            pltpu.VMEM((1, COLS), jnp.float32),
            pltpu.VMEM((1, COLS), jnp.float32),
            pltpu.VMEM((1, 1), jnp.float32),
            pltpu.VMEM((1, COLS), jnp.float32),
            pltpu.VMEM((1, COLS), jnp.float32),
            pltpu.SemaphoreType.DMA((DEPTH,)),
            pltpu.SemaphoreType.DMA((DEPTH,)),
            pltpu.SemaphoreType.DMA((DEPTH,)),
            pltpu.SemaphoreType.DMA((2,)),
        ],
    )(x, m, g, b)
    return out
